# Initial kernel scaffold; baseline (speedup 1.0000x reference)
import jax
import jax.numpy as jnp
from jax.experimental import pallas as pl


def kernel(pred_scores, true_indices):
    raise NotImplementedError("write your pallas kernel here")


# speedup vs baseline: 17.1547x; 17.1547x over previous
import functools

import jax
import jax.numpy as jnp
from jax import lax
from jax.experimental import pallas as pl
from jax.experimental.pallas import tpu as pltpu
from jax.experimental.pallas import tpu_sc as plsc

ROWS = 16384
COLS = 1000
L = 16
NC = 2
NS = 16
NW = NC * NS
RPW = ROWS // NW
G = 16
NG = RPW // G
NCH = COLS // L
TAIL = COLS - NCH * L

_mesh = plsc.VectorSubcoreMesh(core_axis_name="c", subcore_axis_name="s")


@functools.partial(
    pl.kernel,
    mesh=_mesh,
    compiler_params=pltpu.CompilerParams(needs_layout_passes=False),
    out_type=jax.ShapeDtypeStruct((NW, RPW * L), jnp.float32),
    scratch_types=[
        pltpu.VMEM((G * COLS + L,), jnp.float32),
        pltpu.VMEM((G,), jnp.int32),
        pltpu.VMEM((L,), jnp.float32),
        pltpu.VMEM((RPW * L,), jnp.float32),
    ],
)
def _row_sums(pred_hbm, idx_hbm, out_hbm, buf, tbuf, stbuf, sums):
    wid = lax.axis_index("s") * NC + lax.axis_index("c")
    base_row = wid * RPW
    lane = lax.iota(jnp.int32, L)

    def group_body(g, carry):
        row0 = base_row + g * G
        pltpu.sync_copy(pred_hbm.at[pl.ds(row0 * COLS, G * COLS)],
                        buf.at[pl.ds(0, G * COLS)])
        pltpu.sync_copy(idx_hbm.at[pl.ds(row0, G)], tbuf)
        t_vec = tbuf[...]
        stbuf[...] = plsc.load_gather(buf, [lane * COLS + t_vec])

        def row_body(r, carry2):
            rsplat = jnp.full((L,), r, jnp.int32)
            t_b = plsc.load_gather(tbuf, [rsplat])
            s_b = plsc.load_gather(stbuf, [rsplat])
            roff = r * COLS

            def chunk_body(c, acc):
                v = buf[pl.ds(roff + c * L, L)]
                col = c * L + lane
                m = (v < s_b) | ((v == s_b) & (col >= t_b))
                return acc + jnp.where(m, jnp.exp(v - s_b), 0.0)

            acc = lax.fori_loop(0, NCH, chunk_body,
                                jnp.zeros((L,), jnp.float32))
            v = buf[pl.ds(roff + NCH * L, L)]
            col = NCH * L + lane
            m = (v < s_b) | ((v == s_b) & (col >= t_b))
            m = m & (lane < TAIL)
            acc = acc + jnp.where(m, jnp.exp(v - s_b), 0.0)
            sums[pl.ds((g * G + r) * L, L)] = acc
            return carry2

        return lax.fori_loop(0, G, row_body, carry)

    lax.fori_loop(0, NG, group_body, 0)
    pltpu.sync_copy(sums, out_hbm.at[wid])


def _loss_body(s_ref, o_ref):
    row_sums = jnp.sum(s_ref[...], axis=1)
    o_ref[0, 0] = jnp.sum(jnp.log(row_sums)) / ROWS


_loss = pl.pallas_call(
    _loss_body,
    out_shape=jax.ShapeDtypeStruct((1, 1), jnp.float32),
    out_specs=pl.BlockSpec(memory_space=pltpu.SMEM),
)


@jax.jit
def kernel(pred_scores, true_indices):
    if pred_scores.ndim == 1:
        pred_scores = pred_scores[None, :]
    ps = pred_scores.reshape(-1)
    ti = true_indices.reshape(-1).astype(jnp.int32)
    sums = _row_sums(ps, ti)
    return _loss(sums.reshape(ROWS, L))[0, 0]


# speedup vs baseline: 26.0559x; 1.5189x over previous
import functools

import jax
import jax.numpy as jnp
from jax import lax
from jax.experimental import pallas as pl
from jax.experimental.pallas import tpu as pltpu
from jax.experimental.pallas import tpu_sc as plsc

ROWS = 16384
COLS = 1000
L = 16
NC = 2
NS = 16
NW = NC * NS
RPW = ROWS // NW
G = 16
GC = G * COLS
NG = RPW // G
NCH = COLS // L
TAIL = COLS - NCH * L

_mesh = plsc.VectorSubcoreMesh(core_axis_name="c", subcore_axis_name="s")


@functools.partial(
    pl.kernel,
    mesh=_mesh,
    compiler_params=pltpu.CompilerParams(needs_layout_passes=False),
    out_type=jax.ShapeDtypeStruct((NW, RPW * L), jnp.float32),
    scratch_types=[
        pltpu.VMEM((GC + L,), jnp.float32),
        pltpu.VMEM((GC + L,), jnp.float32),
        pltpu.VMEM((RPW,), jnp.int32),
        pltpu.VMEM((L,), jnp.float32),
        pltpu.VMEM((RPW * L,), jnp.float32),
        pltpu.SemaphoreType.DMA,
        pltpu.SemaphoreType.DMA,
    ],
)
def _row_sums(pred_hbm, idx_hbm, out_hbm, buf_a, buf_b, tall, stbuf, sums,
              sem_a, sem_b):
    wid = lax.axis_index("s") * NC + lax.axis_index("c")
    base_row = wid * RPW
    lane = lax.iota(jnp.int32, L)

    pltpu.sync_copy(idx_hbm.at[pl.ds(base_row, RPW)], tall)

    def start_fetch(g, buf, sem):
        src = pred_hbm.at[pl.ds((base_row + g * G) * COLS, GC)]
        pltpu.make_async_copy(src, buf.at[pl.ds(0, GC)], sem).start()

    def wait_fetch(buf, sem):
        src = pred_hbm.at[pl.ds(0, GC)]
        pltpu.make_async_copy(src, buf.at[pl.ds(0, GC)], sem).wait()

    def process_group(g, buf):
        t_vec = tall[pl.ds(g * G, G)]
        stbuf[...] = plsc.load_gather(buf, [lane * COLS + t_vec])

        def row_body(r, carry):
            t_b = plsc.load_gather(tall, [jnp.full((L,), g * G + r, jnp.int32)])
            s_b = plsc.load_gather(stbuf, [jnp.full((L,), r, jnp.int32)])
            bits = plsc.bitcast(s_b, jnp.int32)
            up = jnp.where(s_b > 0.0, bits + 1,
                           jnp.where(s_b < 0.0, bits - 1,
                                     jnp.int32(0x00800000)))
            thr_hi = plsc.bitcast(up, jnp.float32)
            u = t_b - lane
            roff = r * COLS

            acc = jnp.zeros((L,), jnp.float32)
            for c in range(NCH):
                v = buf[pl.ds(roff + c * L, L)]
                thr = jnp.where(c * L >= u, thr_hi, s_b)
                m = v < thr
                acc = acc + jnp.where(m, jnp.exp(v - s_b), 0.0)
            v = buf[pl.ds(roff + NCH * L, L)]
            thr = jnp.where(NCH * L >= u, thr_hi, s_b)
            m = (v < thr) & (lane < TAIL)
            acc = acc + jnp.where(m, jnp.exp(v - s_b), 0.0)
            sums[pl.ds((g * G + r) * L, L)] = acc
            return carry

        lax.fori_loop(0, G, row_body, 0)

    start_fetch(0, buf_a, sem_a)

    def pair_body(i, carry):
        g0 = i * 2
        wait_fetch(buf_a, sem_a)
        start_fetch(g0 + 1, buf_b, sem_b)
        process_group(g0, buf_a)
        wait_fetch(buf_b, sem_b)

        @pl.when(i < NG // 2 - 1)
        def _():
            start_fetch(g0 + 2, buf_a, sem_a)

        process_group(g0 + 1, buf_b)
        return carry

    lax.fori_loop(0, NG // 2, pair_body, 0)
    pltpu.sync_copy(sums, out_hbm.at[wid])


def _loss_body(s_ref, o_ref):
    row_sums = jnp.sum(s_ref[...], axis=1)
    o_ref[0, 0] = jnp.sum(jnp.log(row_sums)) / ROWS


_loss = pl.pallas_call(
    _loss_body,
    out_shape=jax.ShapeDtypeStruct((1, 1), jnp.float32),
    out_specs=pl.BlockSpec(memory_space=pltpu.SMEM),
)


@jax.jit
def kernel(pred_scores, true_indices):
    if pred_scores.ndim == 1:
        pred_scores = pred_scores[None, :]
    ps = pred_scores.reshape(-1)
    ti = true_indices.reshape(-1).astype(jnp.int32)
    sums = _row_sums(ps, ti)
    return _loss(sums.reshape(ROWS, L))[0, 0]
